# Initial kernel scaffold; baseline (speedup 1.0000x reference)
import jax
import jax.numpy as jnp
from jax.experimental import pallas as pl


def kernel(prior0, prior1, sums, k):
    raise NotImplementedError("write your pallas kernel here")


# speedup vs baseline: 12.6989x; 12.6989x over previous
import functools

import jax
import jax.numpy as jnp
from jax import lax
from jax.experimental import pallas as pl
from jax.experimental.pallas import tpu as pltpu

_call = pl.pallas_call

B = 8
T = 2048
SEG = 256
SEGS = T // SEG
M = T * SEGS
K = 64
ROWS = 256
NEG = -1e30
NINF = float("-inf")
IMAX = 2147483647

R128 = (T * T) // 128
OUT_TILE_R = 4096
N_OUT_TILES = R128 // OUT_TILE_R


def _k1_body(p0_ref, p1_ref, s_ref, out_ref):
    s = s_ref[...]
    for b in range(B):
        a = p0_ref[b, :]
        joint = (a[:, None] + p1_ref[b, :][None, :]) + s
        for sg in range(SEGS):
            m = jnp.max(joint[:, sg * SEG:(sg + 1) * SEG], axis=1)
            out_ref[b, sg, :] = m


def _k2_body(bm_ref, bids_ref, cur_ref):
    cur_ref[...] = bm_ref[...]
    it_s = lax.broadcasted_iota(jnp.int32, (B, SEGS, T), 1)
    it_i = lax.broadcasted_iota(jnp.int32, (B, SEGS, T), 2)
    bid3 = it_i * SEGS + it_s

    def step(t, carry):
        cur = cur_ref[...]
        m = jnp.max(jnp.max(cur, axis=2), axis=1)
        m3 = m[:, None, None]
        sel = jnp.where(cur == m3, bid3, IMAX)
        sb = jnp.min(jnp.min(sel, axis=2), axis=1)
        bids_ref[pl.ds(t, 1), :] = sb[None, :]
        cur_ref[...] = jnp.where(bid3 == sb[:, None, None], NINF, cur)
        return carry

    lax.fori_loop(0, K, step, 0)


def _k3_body(s_ref, p0_ref, p1_ref, bsm_ref,
             vals_ref, fidx_ref, cand_ref, fid_ref):
    lane = lax.broadcasted_iota(jnp.int32, (1, 1, SEG), 2)
    for b in range(B):
        def gather(c, carry):
            bid = bsm_ref[c, b]
            i = bid // SEGS
            j0 = pl.multiple_of((bid % SEGS) * SEG, SEG)
            pv = p0_ref[b, i]
            p1row = p1_ref[b, pl.ds(j0, SEG)]
            srow = s_ref[pl.ds(i, 1), pl.ds(j0, SEG)]
            cand_ref[b, pl.ds(c, 1), :] = (pv + p1row)[None, :] + srow
            fid_ref[b, pl.ds(c, 1), :] = (bid * SEG + lane[0]).astype(jnp.int32)
            return carry
        lax.fori_loop(0, K, gather, 0)

    def step(t, carry):
        cand = cand_ref[...]
        fid = fid_ref[...]
        m = jnp.max(jnp.max(cand, axis=2), axis=1)
        m3 = m[:, None, None]
        sel = jnp.where(cand == m3, fid, IMAX)
        fs = jnp.min(jnp.min(sel, axis=2), axis=1)
        vals_ref[pl.ds(t, 1), :] = m[None, :]
        fidx_ref[pl.ds(t, 1), :] = fs[None, :]
        cand_ref[...] = jnp.where(fid == fs[:, None, None], NINF, cand)
        return carry

    lax.fori_loop(0, K, step, 0)


def _k4_body(vsm_ref, fsm_ref, out_ref):
    out_ref[...] = jnp.full((B, OUT_TILE_R, 128), NEG, jnp.float32)
    base_row = pl.program_id(0) * OUT_TILE_R
    lane = lax.broadcasted_iota(jnp.int32, (1, 128), 1)
    for b in range(B):
        def scatter(c, carry):
            f = fsm_ref[c, b]
            r = f // 128 - base_row
            l = f % 128

            @pl.when((r >= 0) & (r < OUT_TILE_R))
            def _():
                cur = out_ref[b, pl.ds(r, 1), :]
                v = vsm_ref[c, b]
                out_ref[b, pl.ds(r, 1), :] = jnp.where(lane == l, v, cur)

            return carry
        lax.fori_loop(0, K, scatter, 0)


def kernel(prior0, prior1, sums, k):
    del k
    blockmax = _call(
        _k1_body,
        grid=(T // ROWS,),
        in_specs=[
            pl.BlockSpec((B, ROWS), lambda it: (0, it)),
            pl.BlockSpec((B, T), lambda it: (0, 0)),
            pl.BlockSpec((ROWS, T), lambda it: (it, 0)),
        ],
        out_specs=pl.BlockSpec((B, SEGS, ROWS), lambda it: (0, 0, it)),
        out_shape=jax.ShapeDtypeStruct((B, SEGS, T), jnp.float32),
    )(prior0, prior1, sums)

    bids = _call(
        _k2_body,
        in_specs=[pl.BlockSpec(memory_space=pltpu.VMEM)],
        out_shape=jax.ShapeDtypeStruct((K, B), jnp.int32),
        scratch_shapes=[pltpu.VMEM((B, SEGS, T), jnp.float32)],
    )(blockmax)

    vals, fidx = _call(
        _k3_body,
        in_specs=[
            pl.BlockSpec(memory_space=pltpu.VMEM),
            pl.BlockSpec(memory_space=pltpu.SMEM),
            pl.BlockSpec(memory_space=pltpu.VMEM),
            pl.BlockSpec(memory_space=pltpu.SMEM),
        ],
        out_shape=(
            jax.ShapeDtypeStruct((K, B), jnp.float32),
            jax.ShapeDtypeStruct((K, B), jnp.int32),
        ),
        scratch_shapes=[
            pltpu.VMEM((B, K, SEG), jnp.float32),
            pltpu.VMEM((B, K, SEG), jnp.int32),
        ],
    )(sums, prior0, prior1, bids)

    out3 = _call(
        _k4_body,
        grid=(N_OUT_TILES,),
        in_specs=[
            pl.BlockSpec(memory_space=pltpu.SMEM),
            pl.BlockSpec(memory_space=pltpu.SMEM),
        ],
        out_specs=pl.BlockSpec((B, OUT_TILE_R, 128), lambda g: (0, g, 0)),
        out_shape=jax.ShapeDtypeStruct((B, R128, 128), jnp.float32),
    )(vals, fidx)

    return out3.reshape(B, T * T)


# speedup vs baseline: 19.2369x; 1.5148x over previous
import functools

import jax
import jax.numpy as jnp
from jax import lax
from jax.experimental import pallas as pl
from jax.experimental.pallas import tpu as pltpu

_call = pl.pallas_call

B = 8
T = 2048
SEG = 256
SEGS = T // SEG
M = T * SEGS
K = 64
ROWS = 256
NEG = -1e30
NINF = float("-inf")
IMAX = 2147483647

R128 = (T * T) // 128
OUT_TILE_R = 4096
N_OUT_TILES = R128 // OUT_TILE_R


def _k1_body(p0_ref, p1_ref, s_ref, out_ref):
    s = s_ref[...]
    for b in range(B):
        a = p0_ref[b, :]
        joint = (a[:, None] + p1_ref[b, :][None, :]) + s
        for sg in range(SEGS):
            m = jnp.max(joint[:, sg * SEG:(sg + 1) * SEG], axis=1)
            out_ref[b, sg, :] = m


def _k2_body(bm_ref, bids_ref, cur_ref):
    cur_ref[...] = bm_ref[...]
    it_s = lax.broadcasted_iota(jnp.int32, (B, SEGS, T), 1)
    it_i = lax.broadcasted_iota(jnp.int32, (B, SEGS, T), 2)
    bid3 = it_i * SEGS + it_s

    def step(t, carry):
        cur = cur_ref[...]
        m = jnp.max(jnp.max(cur, axis=2), axis=1)
        m3 = m[:, None, None]
        sel = jnp.where(cur == m3, bid3, IMAX)
        sb = jnp.min(jnp.min(sel, axis=2), axis=1)
        bids_ref[pl.ds(t, 1), :] = sb[None, :]
        cur_ref[...] = jnp.where(bid3 == sb[:, None, None], NINF, cur)
        return carry

    lax.fori_loop(0, K, step, 0)


def _k3_body(s_ref, p0_ref, p1_ref, bsm_ref,
             vals_ref, fidx_ref, cand_ref, fid_ref):
    lane = lax.broadcasted_iota(jnp.int32, (1, 1, SEG), 2)
    for b in range(B):
        def gather(c, carry):
            bid = bsm_ref[c, b]
            i = bid // SEGS
            j0 = pl.multiple_of((bid % SEGS) * SEG, SEG)
            pv = p0_ref[b, i]
            p1row = p1_ref[b, pl.ds(j0, SEG)]
            srow = s_ref[pl.ds(i, 1), pl.ds(j0, SEG)]
            cand_ref[b, pl.ds(c, 1), :] = (pv + p1row)[None, :] + srow
            fid_ref[b, pl.ds(c, 1), :] = (bid * SEG + lane[0]).astype(jnp.int32)
            return carry
        lax.fori_loop(0, K, gather, 0)

    def step(t, carry):
        cand = cand_ref[...]
        fid = fid_ref[...]
        m = jnp.max(jnp.max(cand, axis=2), axis=1)
        m3 = m[:, None, None]
        sel = jnp.where(cand == m3, fid, IMAX)
        fs = jnp.min(jnp.min(sel, axis=2), axis=1)
        vals_ref[pl.ds(t, 1), :] = m[None, :]
        fidx_ref[pl.ds(t, 1), :] = fs[None, :]
        cand_ref[...] = jnp.where(fid == fs[:, None, None], NINF, cand)
        return carry

    lax.fori_loop(0, K, step, 0)


OUT_TILE = OUT_TILE_R * 128


def _k4_body(vsm_ref, fsm_ref, out_ref):
    out_ref[...] = jnp.full((B, OUT_TILE), NEG, jnp.float32)
    base = pl.program_id(0) * OUT_TILE
    lane = lax.broadcasted_iota(jnp.int32, (128,), 0)
    for b in range(B):
        def scatter(c, carry):
            f = fsm_ref[c, b]
            o = f - base

            @pl.when((o >= 0) & (o < OUT_TILE))
            def _():
                ob = pl.multiple_of((o // 128) * 128, 128)
                cur = out_ref[b, pl.ds(ob, 128)]
                v = vsm_ref[c, b]
                out_ref[b, pl.ds(ob, 128)] = jnp.where(lane == o % 128, v, cur)

            return carry
        lax.fori_loop(0, K, scatter, 0)


def kernel(prior0, prior1, sums, k):
    del k
    blockmax = _call(
        _k1_body,
        grid=(T // ROWS,),
        in_specs=[
            pl.BlockSpec((B, ROWS), lambda it: (0, it)),
            pl.BlockSpec((B, T), lambda it: (0, 0)),
            pl.BlockSpec((ROWS, T), lambda it: (it, 0)),
        ],
        out_specs=pl.BlockSpec((B, SEGS, ROWS), lambda it: (0, 0, it)),
        out_shape=jax.ShapeDtypeStruct((B, SEGS, T), jnp.float32),
    )(prior0, prior1, sums)

    bids = _call(
        _k2_body,
        in_specs=[pl.BlockSpec(memory_space=pltpu.VMEM)],
        out_shape=jax.ShapeDtypeStruct((K, B), jnp.int32),
        scratch_shapes=[pltpu.VMEM((B, SEGS, T), jnp.float32)],
    )(blockmax)

    vals, fidx = _call(
        _k3_body,
        in_specs=[
            pl.BlockSpec(memory_space=pltpu.VMEM),
            pl.BlockSpec(memory_space=pltpu.SMEM),
            pl.BlockSpec(memory_space=pltpu.VMEM),
            pl.BlockSpec(memory_space=pltpu.SMEM),
        ],
        out_shape=(
            jax.ShapeDtypeStruct((K, B), jnp.float32),
            jax.ShapeDtypeStruct((K, B), jnp.int32),
        ),
        scratch_shapes=[
            pltpu.VMEM((B, K, SEG), jnp.float32),
            pltpu.VMEM((B, K, SEG), jnp.int32),
        ],
    )(sums, prior0, prior1, bids)

    return _call(
        _k4_body,
        grid=(N_OUT_TILES,),
        in_specs=[
            pl.BlockSpec(memory_space=pltpu.SMEM),
            pl.BlockSpec(memory_space=pltpu.SMEM),
        ],
        out_specs=pl.BlockSpec((B, OUT_TILE), lambda g: (0, g)),
        out_shape=jax.ShapeDtypeStruct((B, T * T), jnp.float32),
    )(vals, fidx)
